# baseline (speedup 1.0000x reference)
import functools

import jax
import jax.numpy as jnp
from jax import lax
from jax.experimental import pallas as pl
from jax.experimental.pallas import tpu as pltpu
from jax.experimental.pallas import tpu_sc as plsc

BLK = 512
FF_SPLIT = 2
GCH = 40
CCH = 8
NW = 32
LANES = 16


def _route_body(x_ref, gw_ref, gb_ref, d0_ref, d1_ref, cnt_ref,
                acc, poff_s, run_s, e1_s, e2_s):
    p = pl.program_id(0)
    j = pl.program_id(1)
    nb = pl.num_programs(1)
    n = x_ref.shape[0]
    l = gw_ref.shape[1]
    ii = lax.broadcasted_iota(jnp.int32, (n, l), 1)

    @pl.when(p == 0)
    def _():
        logits = jnp.dot(x_ref[...], gw_ref[...],
                         preferred_element_type=jnp.float32)
        logits = logits + gb_ref[...]
        m1v = jnp.max(logits, axis=1, keepdims=True)
        e1 = jnp.min(jnp.where(logits >= m1v, ii, l), axis=1, keepdims=True)
        l2 = jnp.where(ii == e1, -jnp.inf, logits)
        m2v = jnp.max(l2, axis=1, keepdims=True)
        e2 = jnp.min(jnp.where(l2 >= m2v, ii, l), axis=1, keepdims=True)
        e1_s[pl.ds(j * n, n), :] = e1
        e2_s[pl.ds(j * n, n), :] = e2
        m = (ii == e1).astype(jnp.int32) + (ii == e2).astype(jnp.int32)
        csum = jnp.sum(m, axis=0, keepdims=True)
        acc[...] = jnp.where(j == 0, csum, acc[...] + csum)

        @pl.when(j == nb - 1)
        def _():
            cnt_ref[...] = acc[...]

    @pl.when(p == 1)
    def _():
        @pl.when(j == 0)
        def _():
            pc = ((acc[...] + (BLK - 1)) // BLK * BLK).astype(jnp.float32)
            r2 = lax.broadcasted_iota(jnp.int32, (l, l), 0)
            c2 = lax.broadcasted_iota(jnp.int32, (l, l), 1)
            ut = (r2 < c2).astype(jnp.float32)
            poff_s[...] = jnp.dot(pc, ut, preferred_element_type=jnp.float32)

        e1 = e1_s[pl.ds(j * n, n), :]
        e2 = e2_s[pl.ds(j * n, n), :]
        m1 = (ii == e1).astype(jnp.float32)
        m2 = (ii == e2).astype(jnp.float32)
        m = m1 + m2
        r = lax.broadcasted_iota(jnp.int32, (n, n), 0)
        c = lax.broadcasted_iota(jnp.int32, (n, n), 1)
        lt = (c < r).astype(jnp.float32)
        cexcl = jnp.dot(lt, m, preferred_element_type=jnp.float32)
        run = jnp.where(j == 0, jnp.zeros_like(run_s[...]), run_s[...])
        base = cexcl + run + poff_s[...]
        d0_ref[...] = jnp.sum(base * m1, axis=1, keepdims=True).astype(jnp.int32)
        d1_ref[...] = jnp.sum(base * m2, axis=1, keepdims=True).astype(jnp.int32)
        run_s[...] = run + jnp.sum(m, axis=0, keepdims=True)


def _ffn_body_a(ebk_ref, nbu_ref, xs_ref, w1_ref, b1_ref, w2_ref, y_ref):
    j = pl.program_id(0)

    @pl.when(j < nbu_ref[0])
    def _():
        h = jnp.dot(xs_ref[...], w1_ref[0], preferred_element_type=jnp.float32)
        h = jnp.maximum(h + b1_ref[0], 0.0)
        y_ref[...] = jnp.dot(h, w2_ref[0], preferred_element_type=jnp.float32)


def _ffn_body_b(ebk_ref, nbu_ref, xs_ref, w1_ref, b1_ref, w2_ref, b2_ref, y_ref):
    j = pl.program_id(0)

    @pl.when(j < nbu_ref[0])
    def _():
        h = jnp.dot(xs_ref[...], w1_ref[0], preferred_element_type=jnp.float32)
        h = jnp.maximum(h + b1_ref[0], 0.0)
        y = jnp.dot(h, w2_ref[0], preferred_element_type=jnp.float32)
        y_ref[...] = y + b2_ref[0]


def _make_dispatch(T, P, H):
    tok_pw = T // NW
    nch = tok_pw // CCH
    mesh = plsc.VectorSubcoreMesh(core_axis_name="c", subcore_axis_name="s")

    @functools.partial(
        pl.kernel,
        out_type=jax.ShapeDtypeStruct((P, H), jnp.float32),
        mesh=mesh,
        scratch_types=[
            pltpu.VMEM((nch, CCH), jnp.int32),
            pltpu.VMEM((nch, CCH), jnp.int32),
            pltpu.VMEM((CCH, H), jnp.float32),
            pltpu.VMEM((CCH, H), jnp.float32),
            pltpu.SemaphoreType.DMA,
            pltpu.SemaphoreType.DMA,
            pltpu.SemaphoreType.DMA,
            pltpu.SemaphoreType.DMA,
        ],
    )
    def dispatch_k(xf_hbm, dd_hbm, xs_hbm, idx0_v, idx1_v, buf_a, buf_b,
                   la, lb, wa, wb):
        wid = lax.axis_index("s") * 2 + lax.axis_index("c")
        tb = wid * tok_pw
        bufs = (buf_a, buf_b)
        lsems = (la, lb)
        wsems = (wa, wb)
        pltpu.sync_copy(dd_hbm.at[0, pl.ds(wid * nch, nch)], idx0_v)
        pltpu.sync_copy(dd_hbm.at[1, pl.ds(wid * nch, nch)], idx1_v)

        def load(c, b):
            return pltpu.async_copy(
                xf_hbm.at[pl.ds(tb + c * CCH, CCH)], bufs[b], lsems[b]
            )

        def flush(c, b):
            return [
                pltpu.async_copy(bufs[b], xs_hbm.at[idx0_v.at[c]], wsems[b]),
                pltpu.async_copy(bufs[b], xs_hbm.at[idx1_v.at[c]], wsems[b]),
            ]

        lhs = [None, None]
        whs = [None, None]
        lhs[0] = load(0, 0)
        for c in range(nch):
            b = c % 2
            nb = (c + 1) % 2
            if c + 1 < nch:
                if whs[nb] is not None:
                    for h in whs[nb]:
                        h.wait()
                lhs[nb] = load(c + 1, nb)
            lhs[b].wait()
            whs[b] = flush(c, b)
        for hs in whs:
            if hs is not None:
                for h in hs:
                    h.wait()

    return dispatch_k


def _make_combine(T, H, with_part):
    tok_pw = T // NW
    nch = tok_pw // CCH
    nbuf = 3 if with_part else 2
    mesh = plsc.VectorSubcoreMesh(core_axis_name="c", subcore_axis_name="s")

    def build(body):
        return functools.partial(
            pl.kernel,
            out_type=jax.ShapeDtypeStruct((T, H), jnp.float32),
            mesh=mesh,
            scratch_types=[
                pltpu.VMEM((2, tok_pw), jnp.int32),
                [pltpu.VMEM((CCH, H), jnp.float32)] * nbuf,
                [pltpu.VMEM((CCH, H), jnp.float32)] * nbuf,
                [pltpu.VMEM((CCH, H), jnp.float32)] * nbuf,
                pltpu.SemaphoreType.DMA,
                pltpu.SemaphoreType.DMA,
                pltpu.SemaphoreType.DMA,
                pltpu.SemaphoreType.DMA,
                pltpu.SemaphoreType.DMA,
                pltpu.SemaphoreType.DMA,
            ],
        )(body)

    def combine_common(y_hbm, dd_hbm, part_hbm, out_hbm,
                       idx_v, bufs_a, bufs_b, bufs_c, ga, gb, gc, wa, wb, wc):
        wid = lax.axis_index("s") * 2 + lax.axis_index("c")
        tb = wid * tok_pw
        bufs = (bufs_a, bufs_b, bufs_c)
        gsems = (ga, gb, gc)
        wsems = (wa, wb, wc)
        for r in range(2):
            pltpu.sync_copy(dd_hbm.at[r, pl.ds(tb, tok_pw)], idx_v.at[r])

        def fire(c):
            b = c % 3
            hs = [
                pltpu.async_copy(
                    y_hbm.at[idx_v.at[r, pl.ds(c * CCH, CCH)]],
                    bufs[b][r], gsems[b],
                )
                for r in range(2)
            ]
            if with_part:
                hs.append(pltpu.async_copy(
                    part_hbm.at[pl.ds(tb + c * CCH, CCH)], bufs[b][2], gsems[b]
                ))
            return hs

        ghs = [None, None, None]
        whs = [None, None, None]
        ghs[0] = fire(0)
        ghs[1] = fire(1)
        for c in range(nch):
            b = c % 3
            if c + 2 < nch:
                nb = (c + 2) % 3
                if whs[nb] is not None:
                    whs[nb].wait()
                ghs[nb] = fire(c + 2)
            for h in ghs[b]:
                h.wait()
            bb = bufs[b]

            def add_block(i, carry):
                sl = pl.ds(i * LANES, LANES)
                for row in range(CCH):
                    s = bb[0][row, sl] + bb[1][row, sl]
                    if with_part:
                        s = s + bb[2][row, sl]
                    bb[0][row, sl] = s
                return carry

            lax.fori_loop(0, H // LANES, add_block, 0)
            whs[b] = pltpu.async_copy(
                bb[0], out_hbm.at[pl.ds(tb + c * CCH, CCH)], wsems[b]
            )
        for hs in whs:
            if hs is not None:
                hs.wait()

    if with_part:
        def combine_k(y_hbm, dd_hbm, part_hbm, out_hbm, idx_v,
                      bufs_a, bufs_b, bufs_c, ga, gb, gc, wa, wb, wc):
            combine_common(y_hbm, dd_hbm, part_hbm, out_hbm, idx_v,
                           bufs_a, bufs_b, bufs_c, ga, gb, gc, wa, wb, wc)
    else:
        def combine_k(y_hbm, dd_hbm, out_hbm, idx_v,
                      bufs_a, bufs_b, bufs_c, ga, gb, gc, wa, wb, wc):
            combine_common(y_hbm, dd_hbm, None, out_hbm, idx_v,
                           bufs_a, bufs_b, bufs_c, ga, gb, gc, wa, wb, wc)

    return build(combine_k)


def kernel(x, gate_W, gate_b, W1, b1, W2, b2):
    Bq, S, H = x.shape
    E = gate_W.shape[1]
    FF = W1.shape[2]
    T = Bq * S
    TOPK = 2
    A = TOPK * T
    P = A + E * BLK
    NB = P // BLK
    f32 = jnp.float32

    xf = x.reshape(T, H)

    gwp = jnp.pad(gate_W.astype(f32), ((0, 0), (0, 128 - E)))
    gbp = jnp.concatenate(
        [gate_b.astype(f32), jnp.full((128 - E,), -1e30, f32)]
    ).reshape(1, 128)
    gblk = 256
    d0, d1, cnt = pl.pallas_call(
        _route_body,
        grid=(2, T // gblk),
        in_specs=[
            pl.BlockSpec((gblk, H), lambda p, i: ((1 - p) * i, 0)),
            pl.BlockSpec((H, 128), lambda p, i: (0, 0)),
            pl.BlockSpec((1, 128), lambda p, i: (0, 0)),
        ],
        out_specs=[
            pl.BlockSpec((gblk, 1), lambda p, i: (i, 0)),
            pl.BlockSpec((gblk, 1), lambda p, i: (i, 0)),
            pl.BlockSpec((1, 128), lambda p, i: (0, 0)),
        ],
        out_shape=[
            jax.ShapeDtypeStruct((T, 1), jnp.int32),
            jax.ShapeDtypeStruct((T, 1), jnp.int32),
            jax.ShapeDtypeStruct((1, 128), jnp.int32),
        ],
        scratch_shapes=[
            pltpu.VMEM((1, 128), jnp.int32),
            pltpu.VMEM((1, 128), f32),
            pltpu.VMEM((1, 128), f32),
            pltpu.VMEM((T, 1), jnp.int32),
            pltpu.VMEM((T, 1), jnp.int32),
        ],
    )(xf, gwp, gbp)

    counts = cnt[0, :E]
    pc = ((counts + BLK - 1) // BLK) * BLK
    poff = jnp.concatenate(
        [jnp.zeros((1,), jnp.int32), jnp.cumsum(pc).astype(jnp.int32)]
    )
    starts = jnp.arange(NB, dtype=jnp.int32) * BLK
    ebk = jnp.clip(
        jnp.searchsorted(poff, starts, side="right").astype(jnp.int32) - 1,
        0, E - 1,
    )
    nbu = (poff[E] // BLK).reshape(1).astype(jnp.int32)

    d01 = jnp.concatenate([d0.reshape(1, T), d1.reshape(1, T)], axis=0)
    dd3 = d01.reshape(2, T // CCH, CCH)
    xs = _make_dispatch(T, P, H)(xf, dd3)

    ffh = FF // FF_SPLIT
    b1r = b1.astype(f32).reshape(E, 1, FF)
    b2r = b2.astype(f32).reshape(E, 1, H)
    W1f = W1.astype(f32)
    W2f = W2.astype(f32)

    def ffn_pass(body, f, extra):
        return pl.pallas_call(
            body,
            grid_spec=pltpu.PrefetchScalarGridSpec(
                num_scalar_prefetch=2,
                grid=(NB,),
                in_specs=[
                    pl.BlockSpec((BLK, H), lambda j, ebk, nbu: (j, 0)),
                    pl.BlockSpec((1, H, ffh),
                                 lambda j, ebk, nbu, F=f: (ebk[j], 0, F)),
                    pl.BlockSpec((1, 1, ffh),
                                 lambda j, ebk, nbu, F=f: (ebk[j], 0, F)),
                    pl.BlockSpec((1, ffh, H),
                                 lambda j, ebk, nbu, F=f: (ebk[j], F, 0)),
                ] + ([pl.BlockSpec((1, 1, H), lambda j, ebk, nbu: (ebk[j], 0, 0))]
                     if extra else []),
                out_specs=pl.BlockSpec((BLK, H), lambda j, ebk, nbu: (j, 0)),
            ),
            out_shape=jax.ShapeDtypeStruct((P, H), f32),
        )

    y0 = ffn_pass(_ffn_body_a, 0, False)(ebk, nbu, xs, W1f, b1r, W2f)
    part = _make_combine(T, H, False)(y0, d01)
    y1 = ffn_pass(_ffn_body_b, 1, True)(ebk, nbu, xs, W1f, b1r, W2f, b2r)
    out = _make_combine(T, H, True)(y1, d01, part)
    return out.reshape(Bq, S, H)


# speedup vs baseline: 1.0161x; 1.0161x over previous
import functools

import jax
import jax.numpy as jnp
from jax import lax
from jax.experimental import pallas as pl
from jax.experimental.pallas import tpu as pltpu
from jax.experimental.pallas import tpu_sc as plsc

BLK = 512
FF_SPLIT = 2
GCH = 40
CCH = 8
NW = 32
LANES = 16


def _route_body(x_ref, gw_ref, gb_ref, d0_ref, d1_ref, cnt_ref,
                acc, poff_s, run_s, e1_s, e2_s):
    p = pl.program_id(0)
    j = pl.program_id(1)
    nb = pl.num_programs(1)
    n = x_ref.shape[0]
    l = gw_ref.shape[1]
    ii = lax.broadcasted_iota(jnp.int32, (n, l), 1)

    @pl.when(p == 0)
    def _():
        logits = jnp.dot(x_ref[...], gw_ref[...],
                         preferred_element_type=jnp.float32)
        logits = logits + gb_ref[...]
        m1v = jnp.max(logits, axis=1, keepdims=True)
        e1 = jnp.min(jnp.where(logits >= m1v, ii, l), axis=1, keepdims=True)
        l2 = jnp.where(ii == e1, -jnp.inf, logits)
        m2v = jnp.max(l2, axis=1, keepdims=True)
        e2 = jnp.min(jnp.where(l2 >= m2v, ii, l), axis=1, keepdims=True)
        e1_s[pl.ds(j * n, n), :] = e1
        e2_s[pl.ds(j * n, n), :] = e2
        m = (ii == e1).astype(jnp.int32) + (ii == e2).astype(jnp.int32)
        csum = jnp.sum(m, axis=0, keepdims=True)
        acc[...] = jnp.where(j == 0, csum, acc[...] + csum)

        @pl.when(j == nb - 1)
        def _():
            cnt_ref[...] = acc[...]

    @pl.when(p == 1)
    def _():
        @pl.when(j == 0)
        def _():
            pc = ((acc[...] + (BLK - 1)) // BLK * BLK).astype(jnp.float32)
            r2 = lax.broadcasted_iota(jnp.int32, (l, l), 0)
            c2 = lax.broadcasted_iota(jnp.int32, (l, l), 1)
            ut = (r2 < c2).astype(jnp.float32)
            poff_s[...] = jnp.dot(pc, ut, preferred_element_type=jnp.float32)

        e1 = e1_s[pl.ds(j * n, n), :]
        e2 = e2_s[pl.ds(j * n, n), :]
        m1 = (ii == e1).astype(jnp.float32)
        m2 = (ii == e2).astype(jnp.float32)
        m = m1 + m2
        r = lax.broadcasted_iota(jnp.int32, (n, n), 0)
        c = lax.broadcasted_iota(jnp.int32, (n, n), 1)
        lt = (c < r).astype(jnp.float32)
        cexcl = jnp.dot(lt, m, preferred_element_type=jnp.float32)
        run = jnp.where(j == 0, jnp.zeros_like(run_s[...]), run_s[...])
        base = cexcl + run + poff_s[...]
        d0_ref[...] = jnp.sum(base * m1, axis=1, keepdims=True).astype(jnp.int32)
        d1_ref[...] = jnp.sum(base * m2, axis=1, keepdims=True).astype(jnp.int32)
        run_s[...] = run + jnp.sum(m, axis=0, keepdims=True)


def _ffn_body(ebk_ref, nbu_ref, xs_ref, w1_ref, b1_ref, w2_ref, b2_ref, y_ref):
    ffh = pl.program_id(0)
    j = pl.program_id(1)

    @pl.when(j < nbu_ref[0])
    def _():
        h = jnp.dot(xs_ref[...], w1_ref[0], preferred_element_type=jnp.float32)
        h = jnp.maximum(h + b1_ref[0], 0.0)
        y = jnp.dot(h, w2_ref[0], preferred_element_type=jnp.float32)
        y = y + jnp.where(ffh == 1, 1.0, 0.0) * b2_ref[0]
        y_ref[0] = y


def _make_dispatch(T, P, H):
    tok_pw = T // NW
    nch = tok_pw // CCH
    mesh = plsc.VectorSubcoreMesh(core_axis_name="c", subcore_axis_name="s")

    @functools.partial(
        pl.kernel,
        out_type=jax.ShapeDtypeStruct((P, H), jnp.float32),
        mesh=mesh,
        scratch_types=[
            pltpu.VMEM((nch, CCH), jnp.int32),
            pltpu.VMEM((nch, CCH), jnp.int32),
            pltpu.VMEM((CCH, H), jnp.float32),
            pltpu.VMEM((CCH, H), jnp.float32),
            pltpu.SemaphoreType.DMA,
            pltpu.SemaphoreType.DMA,
            pltpu.SemaphoreType.DMA,
            pltpu.SemaphoreType.DMA,
        ],
    )
    def dispatch_k(xf_hbm, dd_hbm, xs_hbm, idx0_v, idx1_v, buf_a, buf_b,
                   la, lb, wa, wb):
        wid = lax.axis_index("s") * 2 + lax.axis_index("c")
        tb = wid * tok_pw
        bufs = (buf_a, buf_b)
        lsems = (la, lb)
        wsems = (wa, wb)
        pltpu.sync_copy(dd_hbm.at[0, pl.ds(wid * nch, nch)], idx0_v)
        pltpu.sync_copy(dd_hbm.at[1, pl.ds(wid * nch, nch)], idx1_v)

        def load(c, b):
            return pltpu.async_copy(
                xf_hbm.at[pl.ds(tb + c * CCH, CCH)], bufs[b], lsems[b]
            )

        def flush(c, b):
            return [
                pltpu.async_copy(bufs[b], xs_hbm.at[idx0_v.at[c]], wsems[b]),
                pltpu.async_copy(bufs[b], xs_hbm.at[idx1_v.at[c]], wsems[b]),
            ]

        lhs = [None, None]
        whs = [None, None]
        lhs[0] = load(0, 0)
        for c in range(nch):
            b = c % 2
            nb = (c + 1) % 2
            if c + 1 < nch:
                if whs[nb] is not None:
                    for h in whs[nb]:
                        h.wait()
                lhs[nb] = load(c + 1, nb)
            lhs[b].wait()
            whs[b] = flush(c, b)
        for hs in whs:
            if hs is not None:
                for h in hs:
                    h.wait()

    return dispatch_k


def _make_combine(T, H):
    tok_pw = T // NW
    nch = tok_pw // CCH
    mesh = plsc.VectorSubcoreMesh(core_axis_name="c", subcore_axis_name="s")

    @functools.partial(
        pl.kernel,
        out_type=jax.ShapeDtypeStruct((T, H), jnp.float32),
        mesh=mesh,
        scratch_types=[
            pltpu.VMEM((4, tok_pw), jnp.int32),
            [pltpu.VMEM((CCH, H), jnp.float32)] * 4,
            [pltpu.VMEM((CCH, H), jnp.float32)] * 4,
            [pltpu.VMEM((CCH, H), jnp.float32)] * 4,
            pltpu.SemaphoreType.DMA,
            pltpu.SemaphoreType.DMA,
            pltpu.SemaphoreType.DMA,
            pltpu.SemaphoreType.DMA,
            pltpu.SemaphoreType.DMA,
            pltpu.SemaphoreType.DMA,
        ],
    )
    def combine_k(y_hbm, dall_hbm, out_hbm, idx_v, bufs_a, bufs_b, bufs_c,
                  ga, gb, gc, wa, wb, wc):
        wid = lax.axis_index("s") * 2 + lax.axis_index("c")
        tb = wid * tok_pw
        bufs = (bufs_a, bufs_b, bufs_c)
        gsems = (ga, gb, gc)
        wsems = (wa, wb, wc)
        for r in range(4):
            pltpu.sync_copy(dall_hbm.at[r, pl.ds(tb, tok_pw)], idx_v.at[r])

        def fire(c):
            b = c % 3
            return [
                pltpu.async_copy(
                    y_hbm.at[idx_v.at[r, pl.ds(c * CCH, CCH)]],
                    bufs[b][r], gsems[b],
                )
                for r in range(4)
            ]

        ghs = [None, None, None]
        whs = [None, None, None]
        ghs[0] = fire(0)
        ghs[1] = fire(1)
        for c in range(nch):
            b = c % 3
            if c + 2 < nch:
                nb = (c + 2) % 3
                if whs[nb] is not None:
                    whs[nb].wait()
                ghs[nb] = fire(c + 2)
            for h in ghs[b]:
                h.wait()
            a0, a1, a2, a3 = bufs[b]

            def add_block(i, carry):
                sl = pl.ds(i * LANES, LANES)
                for row in range(CCH):
                    s = (a1[row, sl] + a2[row, sl]) + a3[row, sl]
                    a0[row, sl] = a0[row, sl] + s
                return carry

            lax.fori_loop(0, H // LANES, add_block, 0)
            whs[b] = pltpu.async_copy(
                a0, out_hbm.at[pl.ds(tb + c * CCH, CCH)], wsems[b]
            )
        for hs in whs:
            if hs is not None:
                hs.wait()

    return combine_k


def kernel(x, gate_W, gate_b, W1, b1, W2, b2):
    Bq, S, H = x.shape
    E = gate_W.shape[1]
    FF = W1.shape[2]
    T = Bq * S
    TOPK = 2
    A = TOPK * T
    P = A + E * BLK
    NB = P // BLK
    f32 = jnp.float32

    xf = x.reshape(T, H)

    gwp = jnp.pad(gate_W.astype(f32), ((0, 0), (0, 128 - E)))
    gbp = jnp.concatenate(
        [gate_b.astype(f32), jnp.full((128 - E,), -1e30, f32)]
    ).reshape(1, 128)
    gblk = 256
    d0, d1, cnt = pl.pallas_call(
        _route_body,
        grid=(2, T // gblk),
        in_specs=[
            pl.BlockSpec((gblk, H), lambda p, i: ((1 - p) * i, 0)),
            pl.BlockSpec((H, 128), lambda p, i: (0, 0)),
            pl.BlockSpec((1, 128), lambda p, i: (0, 0)),
        ],
        out_specs=[
            pl.BlockSpec((gblk, 1), lambda p, i: (i, 0)),
            pl.BlockSpec((gblk, 1), lambda p, i: (i, 0)),
            pl.BlockSpec((1, 128), lambda p, i: (0, 0)),
        ],
        out_shape=[
            jax.ShapeDtypeStruct((T, 1), jnp.int32),
            jax.ShapeDtypeStruct((T, 1), jnp.int32),
            jax.ShapeDtypeStruct((1, 128), jnp.int32),
        ],
        scratch_shapes=[
            pltpu.VMEM((1, 128), jnp.int32),
            pltpu.VMEM((1, 128), f32),
            pltpu.VMEM((1, 128), f32),
            pltpu.VMEM((T, 1), jnp.int32),
            pltpu.VMEM((T, 1), jnp.int32),
        ],
    )(xf, gwp, gbp)

    counts = cnt[0, :E]
    pc = ((counts + BLK - 1) // BLK) * BLK
    poff = jnp.concatenate(
        [jnp.zeros((1,), jnp.int32), jnp.cumsum(pc).astype(jnp.int32)]
    )
    starts = jnp.arange(NB, dtype=jnp.int32) * BLK
    ebk = jnp.clip(
        jnp.searchsorted(poff, starts, side="right").astype(jnp.int32) - 1,
        0, E - 1,
    )
    nbu = (poff[E] // BLK).reshape(1).astype(jnp.int32)

    d01 = jnp.concatenate([d0.reshape(1, T), d1.reshape(1, T)], axis=0)
    dd3 = d01.reshape(2, T // CCH, CCH)
    xs = _make_dispatch(T, P, H)(xf, dd3)

    ffh = FF // FF_SPLIT
    y = pl.pallas_call(
        _ffn_body,
        grid_spec=pltpu.PrefetchScalarGridSpec(
            num_scalar_prefetch=2,
            grid=(FF_SPLIT, NB),
            in_specs=[
                pl.BlockSpec((BLK, H), lambda f, j, ebk, nbu: (j, 0)),
                pl.BlockSpec((1, H, ffh), lambda f, j, ebk, nbu: (ebk[j], 0, f)),
                pl.BlockSpec((1, 1, ffh), lambda f, j, ebk, nbu: (ebk[j], 0, f)),
                pl.BlockSpec((1, ffh, H), lambda f, j, ebk, nbu: (ebk[j], f, 0)),
                pl.BlockSpec((1, 1, H), lambda f, j, ebk, nbu: (ebk[j], 0, 0)),
            ],
            out_specs=pl.BlockSpec((1, BLK, H), lambda f, j, ebk, nbu: (f, j, 0)),
        ),
        out_shape=jax.ShapeDtypeStruct((FF_SPLIT, P, H), f32),
    )(ebk, nbu, xs, W1.astype(f32), b1.astype(f32).reshape(E, 1, FF),
      W2.astype(f32), b2.astype(f32).reshape(E, 1, H))

    dall = jnp.concatenate([d01, d01 + P], axis=0)
    y2 = y.reshape(FF_SPLIT * P, H)
    out = _make_combine(T, H)(y2, dall)
    return out.reshape(Bq, S, H)
